# baseline (speedup 1.0000x reference)
import functools

import jax
import jax.numpy as jnp
from jax import lax
from jax.experimental import pallas as pl
from jax.experimental.pallas import tpu as pltpu
from jax.experimental.pallas import tpu_sc as plsc

VOCAB = 100000
EMBED_DIM = 128
BATCH = 4096
HIST_LEN = 50

_NC = 2
_NS = 16
_NW = _NC * _NS
_ROWS = BATCH * HIST_LEN
_COLS = 128
_CHUNK = 64
_NBUF = 5
_NROUNDS = HIST_LEN // _NBUF

_mesh = plsc.VectorSubcoreMesh(core_axis_name="c", subcore_axis_name="s")


@functools.partial(
    pl.kernel,
    mesh=_mesh,
    out_type=jax.ShapeDtypeStruct((_ROWS, EMBED_DIM), jnp.float32),
    scratch_types=[
        pltpu.VMEM((HIST_LEN, _COLS), jnp.int32),
        pltpu.VMEM((_NBUF, _COLS, EMBED_DIM), jnp.float32),
    ]
    + [pltpu.SemaphoreType.DMA] * (2 * _NBUF),
)
def _emb_lookup(idx_hbm, table_hbm, out_hbm, idx_v, rows_v, *sems):
    gsem = sems[:_NBUF]
    ssem = sems[_NBUF:]
    wid = lax.axis_index("s") * _NC + lax.axis_index("c")
    bbase = wid * _COLS
    pltpu.sync_copy(idx_hbm.at[:, pl.ds(bbase, _COLS)], idx_v)

    def gather(t, s):
        for half in range(2):
            idx = idx_v.at[t, pl.ds(half * _CHUNK, _CHUNK)]
            dst = rows_v.at[s, pl.ds(half * _CHUNK, _CHUNK)]
            pltpu.async_copy(table_hbm.at[idx], dst, gsem[s])

    def gather_wait(s):
        idx = idx_v.at[0, pl.ds(0, _CHUNK)]
        dst = rows_v.at[s, pl.ds(0, _CHUNK)]
        for _ in range(2):
            pltpu.make_async_copy(table_hbm.at[idx], dst, gsem[s]).wait()

    def store(t, s):
        dst = out_hbm.at[pl.ds(t * BATCH + bbase, _COLS)]
        pltpu.async_copy(rows_v.at[s], dst, ssem[s])

    def store_wait(s):
        dst = out_hbm.at[pl.ds(bbase, _COLS)]
        pltpu.make_async_copy(rows_v.at[s], dst, ssem[s]).wait()

    for s in range(_NBUF):
        gather(s, s)

    def round_body(h, carry):
        t0 = _NBUF * h
        for s in range(_NBUF):
            gather_wait(s)
            store(t0 + s, s)
        for s in range(_NBUF):
            store_wait(s)
            gather(t0 + s + _NBUF, s)
        return carry

    lax.fori_loop(0, _NROUNDS - 1, round_body, 0)

    t0 = _NBUF * (_NROUNDS - 1)
    for s in range(_NBUF):
        gather_wait(s)
        store(t0 + s, s)
    for s in range(_NBUF):
        store_wait(s)


def kernel(indices, embedding_matrix):
    out = _emb_lookup(indices.T.astype(jnp.int32), embedding_matrix)
    return out.reshape(HIST_LEN, BATCH, EMBED_DIM).transpose(1, 0, 2)


# speedup vs baseline: 1.0286x; 1.0286x over previous
import functools

import jax
import jax.numpy as jnp
from jax import lax
from jax.experimental import pallas as pl
from jax.experimental.pallas import tpu as pltpu
from jax.experimental.pallas import tpu_sc as plsc

VOCAB = 100000
EMBED_DIM = 128
BATCH = 4096
HIST_LEN = 50

_NC = 2
_NS = 16
_NW = _NC * _NS
_ROWS = BATCH * HIST_LEN
_COLS = 128
_CHUNK = 64
_NCH = HIST_LEN * (_COLS // _CHUNK)
_NBUF = 10
_NROUNDS = _NCH // _NBUF

_mesh = plsc.VectorSubcoreMesh(core_axis_name="c", subcore_axis_name="s")


@functools.partial(
    pl.kernel,
    mesh=_mesh,
    out_type=jax.ShapeDtypeStruct((_ROWS, EMBED_DIM), jnp.float32),
    scratch_types=[
        pltpu.VMEM((HIST_LEN, _COLS), jnp.int32),
        pltpu.VMEM((_NBUF, _CHUNK, EMBED_DIM), jnp.float32),
    ]
    + [pltpu.SemaphoreType.DMA] * (2 * _NBUF),
)
def _emb_lookup(idx_hbm, table_hbm, out_hbm, idx_v, rows_v, *sems):
    gsem = sems[:_NBUF]
    ssem = sems[_NBUF:]
    wid = lax.axis_index("s") * _NC + lax.axis_index("c")
    bbase = wid * _COLS
    pltpu.sync_copy(idx_hbm.at[:, pl.ds(bbase, _COLS)], idx_v)

    def gather(t, half, s):
        idx = idx_v.at[t, pl.ds(half * _CHUNK, _CHUNK)]
        pltpu.async_copy(table_hbm.at[idx], rows_v.at[s], gsem[s])

    def gather_wait(s):
        idx = idx_v.at[0, pl.ds(0, _CHUNK)]
        pltpu.make_async_copy(table_hbm.at[idx], rows_v.at[s], gsem[s]).wait()

    def store(t, half, s):
        dst = out_hbm.at[pl.ds(t * BATCH + bbase + half * _CHUNK, _CHUNK)]
        pltpu.async_copy(rows_v.at[s], dst, ssem[s])

    def store_wait(s):
        dst = out_hbm.at[pl.ds(bbase, _CHUNK)]
        pltpu.make_async_copy(rows_v.at[s], dst, ssem[s]).wait()

    for s in range(_NBUF):
        gather(s // 2, s % 2, s)

    def round_body(h, carry):
        c0 = _NBUF * h
        for s in range(_NBUF):
            gather_wait(s)
            store((c0 + s) // 2, s % 2, s)
        for s in range(_NBUF):
            store_wait(s)
            gather((c0 + s + _NBUF) // 2, s % 2, s)
        return carry

    lax.fori_loop(0, _NROUNDS - 1, round_body, 0)

    c0 = _NBUF * (_NROUNDS - 1)
    for s in range(_NBUF):
        gather_wait(s)
        store((c0 + s) // 2, s % 2, s)
    for s in range(_NBUF):
        store_wait(s)


def kernel(indices, embedding_matrix):
    out = _emb_lookup(indices.T.astype(jnp.int32), embedding_matrix)
    return out.reshape(HIST_LEN, BATCH, EMBED_DIM).transpose(1, 0, 2)
